# baseline (speedup 1.0000x reference)
import functools

import jax
import jax.numpy as jnp
from jax.experimental import pallas as pl
from jax.experimental.pallas import tpu as pltpu

_B, _T, _D_IN, _G, _D = 8, 2048, 512, 8, 256
_N = _B * _T
_BLK = 256
_GD = _G * _D


def _fused_body(x_ref, wpi_ref, bpi_ref, wsig_ref, bsig_ref, wmu_ref, bmu_ref,
                mu_ref, sig_ref, wmu_s, wsig_s):
    @pl.when(pl.program_id(0) == 0)
    def _cast_weights():
        wmu_s[...] = wmu_ref[...].astype(jnp.bfloat16)
        wsig_s[...] = wsig_ref[...].astype(jnp.bfloat16)

    x = x_ref[...]
    logits = jnp.dot(x, wpi_ref[...], preferred_element_type=jnp.float32)
    logits = logits + bpi_ref[...]
    g = jnp.argmax(logits, axis=1).astype(jnp.int32)

    lane_group = jax.lax.broadcasted_iota(jnp.int32, (_BLK, _GD), 1) // _D
    keep = lane_group == g[:, None]

    xh = x.astype(jnp.bfloat16)
    mu_full = jnp.dot(xh, wmu_s[...], preferred_element_type=jnp.float32)
    mu_full = jnp.where(keep, mu_full + bmu_ref[...], 0.0)
    acc_mu = jnp.zeros((_BLK, _D), jnp.float32)
    for k in range(_G):
        acc_mu = acc_mu + mu_full[:, k * _D:(k + 1) * _D]
    mu_ref[...] = acc_mu

    sig_full = jnp.dot(xh, wsig_s[...], preferred_element_type=jnp.float32)
    sig_full = jnp.where(keep, sig_full + bsig_ref[...], 0.0)
    acc_sig = jnp.zeros((_BLK, _D), jnp.float32)
    for k in range(_G):
        acc_sig = acc_sig + sig_full[:, k * _D:(k + 1) * _D]
    sig_ref[...] = jnp.exp(acc_sig)


@jax.jit
def kernel(x, W_pi, b_pi, W_sigma, b_sigma, W_mu, b_mu):
    xf = x.reshape(_N, _D_IN)
    grid = (_N // _BLK,)
    full = lambda i: (0, 0)
    mu, sig = pl.pallas_call(
        _fused_body,
        grid=grid,
        in_specs=[
            pl.BlockSpec((_BLK, _D_IN), lambda i: (i, 0)),
            pl.BlockSpec((_D_IN, _G), full),
            pl.BlockSpec((_G,), lambda i: (0,)),
            pl.BlockSpec((_D_IN, _GD), full),
            pl.BlockSpec((1, _GD), full),
            pl.BlockSpec((_D_IN, _GD), full),
            pl.BlockSpec((1, _GD), full),
        ],
        out_specs=[
            pl.BlockSpec((_BLK, _D), lambda i: (i, 0)),
            pl.BlockSpec((_BLK, _D), lambda i: (i, 0)),
        ],
        out_shape=[
            jax.ShapeDtypeStruct((_N, _D), jnp.float32),
            jax.ShapeDtypeStruct((_N, _D), jnp.float32),
        ],
        scratch_shapes=[
            pltpu.VMEM((_D_IN, _GD), jnp.bfloat16),
            pltpu.VMEM((_D_IN, _GD), jnp.bfloat16),
        ],
        compiler_params=pltpu.CompilerParams(
            dimension_semantics=("arbitrary",),
        ),
    )(xf, W_pi, b_pi, W_sigma, b_sigma[None, :], W_mu, b_mu[None, :])
    return mu.reshape(_B, _T, _D), sig.reshape(_B, _T, _D)


# speedup vs baseline: 1.2172x; 1.2172x over previous
import functools

import jax
import jax.numpy as jnp
from jax.experimental import pallas as pl
from jax.experimental.pallas import tpu as pltpu

_B, _T, _D_IN, _G, _D = 8, 2048, 512, 8, 256
_N = _B * _T
_BLK = 1024
_GD = _G * _D


def _fused_body(x_ref, wpi_ref, bpi_ref, wsig_ref, bsig_ref, wmu_ref, bmu_ref,
                mu_ref, sig_ref, wmu_s, wsig_s):
    @pl.when(pl.program_id(0) == 0)
    def _cast_weights():
        wmu_s[...] = wmu_ref[...].astype(jnp.bfloat16)
        wsig_s[...] = wsig_ref[...].astype(jnp.bfloat16)

    x = x_ref[...]
    logits = jnp.dot(x, wpi_ref[...], preferred_element_type=jnp.float32)
    logits = logits + bpi_ref[...]
    g = jnp.argmax(logits, axis=1).astype(jnp.int32)

    lane_group = jax.lax.broadcasted_iota(jnp.int32, (_BLK, _GD), 1) // _D
    keep = lane_group == g[:, None]

    xh = x.astype(jnp.bfloat16)
    mu_full = jnp.dot(xh, wmu_s[...], preferred_element_type=jnp.float32)
    mu_full = jnp.where(keep, mu_full + bmu_ref[...], 0.0)
    acc_mu = jnp.zeros((_BLK, _D), jnp.float32)
    for k in range(_G):
        acc_mu = acc_mu + mu_full[:, k * _D:(k + 1) * _D]
    mu_ref[...] = acc_mu

    sig_full = jnp.dot(xh, wsig_s[...], preferred_element_type=jnp.float32)
    sig_full = jnp.where(keep, sig_full + bsig_ref[...], 0.0)
    acc_sig = jnp.zeros((_BLK, _D), jnp.float32)
    for k in range(_G):
        acc_sig = acc_sig + sig_full[:, k * _D:(k + 1) * _D]
    sig_ref[...] = jnp.exp(acc_sig)


@jax.jit
def kernel(x, W_pi, b_pi, W_sigma, b_sigma, W_mu, b_mu):
    xf = x.reshape(_N, _D_IN)
    grid = (_N // _BLK,)
    full = lambda i: (0, 0)
    mu, sig = pl.pallas_call(
        _fused_body,
        grid=grid,
        in_specs=[
            pl.BlockSpec((_BLK, _D_IN), lambda i: (i, 0)),
            pl.BlockSpec((_D_IN, _G), full),
            pl.BlockSpec((_G,), lambda i: (0,)),
            pl.BlockSpec((_D_IN, _GD), full),
            pl.BlockSpec((1, _GD), full),
            pl.BlockSpec((_D_IN, _GD), full),
            pl.BlockSpec((1, _GD), full),
        ],
        out_specs=[
            pl.BlockSpec((_BLK, _D), lambda i: (i, 0)),
            pl.BlockSpec((_BLK, _D), lambda i: (i, 0)),
        ],
        out_shape=[
            jax.ShapeDtypeStruct((_N, _D), jnp.float32),
            jax.ShapeDtypeStruct((_N, _D), jnp.float32),
        ],
        scratch_shapes=[
            pltpu.VMEM((_D_IN, _GD), jnp.bfloat16),
            pltpu.VMEM((_D_IN, _GD), jnp.bfloat16),
        ],
        compiler_params=pltpu.CompilerParams(
            dimension_semantics=("arbitrary",),
        ),
    )(xf, W_pi, b_pi, W_sigma, b_sigma[None, :], W_mu, b_mu[None, :])
    return mu.reshape(_B, _T, _D), sig.reshape(_B, _T, _D)
